# Initial kernel scaffold; baseline (speedup 1.0000x reference)
import functools

import jax
import jax.numpy as jnp
from jax import lax
from jax.experimental import pallas as pl
from jax.experimental.pallas import tpu as pltpu
from jax.experimental.pallas import tpu_sc as plsc

N_NODES = 10000
N_EDGES = 320000
D = 16
K_PROP = 10
ALPHA = 0.1

NS = 16
ET = N_EDGES // NS
CHUNK = 2000
NCHUNK = ET // CHUNK
STRIPE = N_NODES // NS


def _mlp_body(x_ref, w1_ref, b1_ref, w2_ref, b2_ref, o_ref):
    h = jnp.dot(x_ref[...], w1_ref[...], preferred_element_type=jnp.float32)
    h = jnp.maximum(h + b1_ref[...], 0.0)
    o_ref[...] = (
        jnp.dot(h, w2_ref[...], preferred_element_type=jnp.float32) + b2_ref[...]
    )


def _mlp(x, W1, b1, W2, b2):
    n, d_in = x.shape
    hid = W1.shape[1]
    blk = 1000
    return pl.pallas_call(
        _mlp_body,
        grid=(n // blk,),
        in_specs=[
            pl.BlockSpec((blk, d_in), lambda i: (i, 0)),
            pl.BlockSpec((d_in, hid), lambda i: (0, 0)),
            pl.BlockSpec((1, hid), lambda i: (0, 0)),
            pl.BlockSpec((hid, D), lambda i: (0, 0)),
            pl.BlockSpec((1, D), lambda i: (0, 0)),
        ],
        out_specs=pl.BlockSpec((blk, D), lambda i: (i, 0)),
        out_shape=jax.ShapeDtypeStruct((n, D), jnp.float32),
    )(x, W1, b1.reshape(1, hid), W2, b2.reshape(1, D))


def _rsqrt16(d):
    bits = plsc.bitcast(d, jnp.int32)
    y = plsc.bitcast(jnp.int32(0x5F3759DF) - (bits >> 1), jnp.float32)
    for _ in range(3):
        y = y * (1.5 - 0.5 * d * y * y)
    return y


def _appnp_body(h0_hbm, src_hbm, dst_hbm, out_hbm, g_hbm,
                src_v, dst_v, rows_v, dis_v, h0_v, c1_v, c2_v, s_sp, sem):
    t = lax.axis_index("s")
    base = t * STRIPE
    stripe = pl.ds(base, STRIPE)
    r625 = pl.ds(0, STRIPE)

    pltpu.sync_copy(src_hbm.at[t], src_v)
    pltpu.sync_copy(dst_hbm.at[t], dst_v)

    def fill_one(i, _):
        rows_v[i, :] = jnp.full((D,), 1.0, jnp.float32)
        return ()
    lax.fori_loop(0, CHUNK, fill_one, ())

    pltpu.sync_copy(rows_v.at[r625], s_sp.at[stripe])
    plsc.subcore_barrier()

    def deg_chunk(j, _):
        pltpu.sync_copy(rows_v, s_sp.at[dst_v.at[j]], add=True)
        return ()
    lax.fori_loop(0, NCHUNK, deg_chunk, ())
    plsc.subcore_barrier()

    pltpu.sync_copy(s_sp.at[stripe], rows_v.at[r625])
    pltpu.sync_copy(h0_hbm.at[stripe], h0_v)

    def coeffs(i, _):
        y = _rsqrt16(rows_v[i, :])
        h0r = h0_v[i, :]
        dis_v[i, :] = y
        c1_v[i, :] = (1.0 - ALPHA) * y * y
        c2_v[i, :] = ALPHA * y * h0r
        rows_v[i, :] = y * h0r
        return ()
    lax.fori_loop(0, STRIPE, coeffs, ())

    pltpu.sync_copy(rows_v.at[r625], g_hbm.at[stripe])
    pltpu.sync_copy(rows_v.at[r625], s_sp.at[stripe])
    plsc.subcore_barrier()

    def edge_pass(j, _):
        pltpu.async_copy(g_hbm.at[src_v.at[j]], rows_v, sem).wait()
        pltpu.sync_copy(rows_v, s_sp.at[dst_v.at[j]], add=True)
        return ()

    def update(i, _):
        rows_v[i, :] = c1_v[i, :] * rows_v[i, :] + c2_v[i, :]
        return ()

    def final(i, _):
        rows_v[i, :] = ((1.0 - ALPHA) * dis_v[i, :] * rows_v[i, :]
                        + ALPHA * h0_v[i, :])
        return ()

    for k in range(K_PROP):
        lax.fori_loop(0, NCHUNK, edge_pass, ())
        plsc.subcore_barrier()

        pltpu.sync_copy(s_sp.at[stripe], rows_v.at[r625])
        if k < K_PROP - 1:
            lax.fori_loop(0, STRIPE, update, ())
            pltpu.sync_copy(rows_v.at[r625], g_hbm.at[stripe])
            pltpu.sync_copy(rows_v.at[r625], s_sp.at[stripe])
            plsc.subcore_barrier()
        else:
            lax.fori_loop(0, STRIPE, final, ())
            pltpu.sync_copy(rows_v.at[r625], out_hbm.at[stripe])


@jax.jit
def _run(x, src, dst, W1, b1, W2, b2):
    h0 = _mlp(x, W1, b1, W2, b2)

    mesh = plsc.VectorSubcoreMesh(
        core_axis_name="c", subcore_axis_name="s", num_cores=1
    )
    appnp = pl.kernel(
        _appnp_body,
        out_type=(
            jax.ShapeDtypeStruct((N_NODES, D), jnp.float32),
            jax.ShapeDtypeStruct((N_NODES, D), jnp.float32),
        ),
        mesh=mesh,
        scratch_types=[
            pltpu.VMEM((NCHUNK, CHUNK), jnp.int32),
            pltpu.VMEM((NCHUNK, CHUNK), jnp.int32),
            pltpu.VMEM((CHUNK, D), jnp.float32),
            pltpu.VMEM((STRIPE, D), jnp.float32),
            pltpu.VMEM((STRIPE, D), jnp.float32),
            pltpu.VMEM((STRIPE, D), jnp.float32),
            pltpu.VMEM((STRIPE, D), jnp.float32),
            pltpu.VMEM_SHARED((N_NODES, D), jnp.float32),
            pltpu.SemaphoreType.DMA,
        ],
    )
    out, _ = appnp(h0, src, dst)
    return out


def kernel(x, edge_index, epoch, W1, b1, W2, b2):
    src = edge_index[0].astype(jnp.int32).reshape(NS, NCHUNK, CHUNK)
    dst = edge_index[1].astype(jnp.int32).reshape(NS, NCHUNK, CHUNK)
    return _run(x, src, dst, W1, b1, W2, b2)


# speedup vs baseline: 43.3872x; 43.3872x over previous
import jax
import jax.numpy as jnp
from jax import lax
from jax.experimental import pallas as pl
from jax.experimental.pallas import tpu as pltpu
from jax.experimental.pallas import tpu_sc as plsc

N_NODES = 10000
N_EDGES = 320000
D = 16
K_PROP = 10
ALPHA = 0.1

NS = 16
ET = N_EDGES // NS
CHUNK = 2000
NCHUNK = ET // CHUNK
N_PAD = 10240
STRIPE = N_PAD // NS

_MESH = plsc.VectorSubcoreMesh(
    core_axis_name="c", subcore_axis_name="s", num_cores=1
)
_SC_PARAMS = pltpu.CompilerParams(use_tc_tiling_on_sc=False)


def _deg_body(dst_hbm, deg_hbm, dst_v, ones_v, s_sp):
    t = lax.axis_index("s")
    stripe = pl.ds(t * STRIPE, STRIPE)

    pltpu.sync_copy(dst_hbm.at[t], dst_v)

    def fill_one(i, _):
        ones_v[i, :] = jnp.full((D,), 1.0, jnp.float32)
        return ()
    lax.fori_loop(0, CHUNK, fill_one, ())

    pltpu.sync_copy(ones_v.at[pl.ds(0, STRIPE)], s_sp.at[stripe])
    plsc.subcore_barrier()

    def deg_chunk(j, _):
        pltpu.sync_copy(ones_v, s_sp.at[dst_v.at[j]], add=True)
        return ()
    lax.fori_loop(0, NCHUNK, deg_chunk, ())
    plsc.subcore_barrier()

    pltpu.sync_copy(s_sp.at[stripe], deg_hbm.at[stripe])


def _tc_body(x_ref, w1_ref, b1_ref, w2_ref, b2_ref, deg_ref,
             g0_ref, c1_ref, c2_ref, c3_ref, c4_ref):
    h = jnp.dot(x_ref[...], w1_ref[...], preferred_element_type=jnp.float32)
    h = jnp.maximum(h + b1_ref[...], 0.0)
    h0 = jnp.dot(h, w2_ref[...], preferred_element_type=jnp.float32) + b2_ref[...]
    dis = lax.rsqrt(deg_ref[...])
    g0_ref[...] = dis * h0
    c1_ref[...] = (1.0 - ALPHA) * dis * dis
    c2_ref[...] = ALPHA * dis * h0
    c3_ref[...] = (1.0 - ALPHA) * dis
    c4_ref[...] = ALPHA * h0


def _tc_stage(x, W1, b1, W2, b2, deg_b):
    n, d_in = x.shape
    hid = W1.shape[1]
    blk = 1024
    full = lambda shape: pl.BlockSpec(shape, lambda i: (0, 0))
    row = pl.BlockSpec((blk, D), lambda i: (i, 0))
    out_sds = jax.ShapeDtypeStruct((n, D), jnp.float32)
    return pl.pallas_call(
        _tc_body,
        grid=(n // blk,),
        in_specs=[
            pl.BlockSpec((blk, d_in), lambda i: (i, 0)),
            full((d_in, hid)),
            full((1, hid)),
            full((hid, D)),
            full((1, D)),
            row,
        ],
        out_specs=(row, row, row, row, row),
        out_shape=(out_sds,) * 5,
    )(x, W1, b1.reshape(1, hid), W2, b2.reshape(1, D), deg_b)


def _appnp_body(g0_hbm, c1_hbm, c2_hbm, c3_hbm, c4_hbm, src_hbm, dst_hbm,
                out_hbm, g_hbm,
                src_v, dst_v, rows_v, c1_v, c2_v, s_sp, sem):
    t = lax.axis_index("s")
    stripe = pl.ds(t * STRIPE, STRIPE)
    rS = pl.ds(0, STRIPE)

    pltpu.sync_copy(src_hbm.at[t], src_v)
    pltpu.sync_copy(dst_hbm.at[t], dst_v)
    pltpu.sync_copy(c1_hbm.at[stripe], c1_v)
    pltpu.sync_copy(c2_hbm.at[stripe], c2_v)

    pltpu.sync_copy(g0_hbm.at[stripe], rows_v.at[rS])
    pltpu.sync_copy(rows_v.at[rS], g_hbm.at[stripe])
    pltpu.sync_copy(rows_v.at[rS], s_sp.at[stripe])
    plsc.subcore_barrier()

    def edge_pass(j, _):
        pltpu.async_copy(g_hbm.at[src_v.at[j]], rows_v, sem).wait()
        pltpu.sync_copy(rows_v, s_sp.at[dst_v.at[j]], add=True)
        return ()

    def update(i, _):
        rows_v[i, :] = c1_v[i, :] * rows_v[i, :] + c2_v[i, :]
        return ()

    for k in range(K_PROP):
        lax.fori_loop(0, NCHUNK, edge_pass, ())
        plsc.subcore_barrier()

        pltpu.sync_copy(s_sp.at[stripe], rows_v.at[rS])
        if k < K_PROP - 1:
            lax.fori_loop(0, STRIPE, update, ())
            pltpu.sync_copy(rows_v.at[rS], g_hbm.at[stripe])
            pltpu.sync_copy(rows_v.at[rS], s_sp.at[stripe])
            plsc.subcore_barrier()
        else:
            pltpu.sync_copy(c3_hbm.at[stripe], c1_v)
            pltpu.sync_copy(c4_hbm.at[stripe], c2_v)
            lax.fori_loop(0, STRIPE, update, ())
            pltpu.sync_copy(rows_v.at[rS], out_hbm.at[stripe])


@jax.jit
def _run(x, src, dst, W1, b1, W2, b2):
    deg_kernel = pl.kernel(
        _deg_body,
        out_type=jax.ShapeDtypeStruct((N_PAD, D), jnp.float32),
        mesh=_MESH,
        compiler_params=_SC_PARAMS,
        scratch_types=[
            pltpu.VMEM((NCHUNK, CHUNK), jnp.int32),
            pltpu.VMEM((CHUNK, D), jnp.float32),
            pltpu.VMEM_SHARED((N_PAD, D), jnp.float32),
        ],
    )
    deg_b = deg_kernel(dst)

    xp = jnp.concatenate(
        [x, jnp.zeros((N_PAD - N_NODES, x.shape[1]), x.dtype)], axis=0
    )
    g0, c1, c2, c3, c4 = _tc_stage(xp, W1, b1, W2, b2, deg_b)

    appnp = pl.kernel(
        _appnp_body,
        out_type=(
            jax.ShapeDtypeStruct((N_PAD, D), jnp.float32),
            jax.ShapeDtypeStruct((N_PAD, D), jnp.float32),
        ),
        mesh=_MESH,
        compiler_params=_SC_PARAMS,
        scratch_types=[
            pltpu.VMEM((NCHUNK, CHUNK), jnp.int32),
            pltpu.VMEM((NCHUNK, CHUNK), jnp.int32),
            pltpu.VMEM((CHUNK, D), jnp.float32),
            pltpu.VMEM((STRIPE, D), jnp.float32),
            pltpu.VMEM((STRIPE, D), jnp.float32),
            pltpu.VMEM_SHARED((N_PAD, D), jnp.float32),
            pltpu.SemaphoreType.DMA,
        ],
    )
    out, _ = appnp(g0, c1, c2, c3, c4, src, dst)
    return out[:N_NODES]


def kernel(x, edge_index, epoch, W1, b1, W2, b2):
    src = edge_index[0].astype(jnp.int32).reshape(NS, NCHUNK, CHUNK)
    dst = edge_index[1].astype(jnp.int32).reshape(NS, NCHUNK, CHUNK)
    return _run(x, src, dst, W1, b1, W2, b2)


# speedup vs baseline: 48.2066x; 1.1111x over previous
import jax
import jax.numpy as jnp
from jax import lax
from jax.experimental import pallas as pl
from jax.experimental.pallas import tpu as pltpu
from jax.experimental.pallas import tpu_sc as plsc

N_NODES = 10000
N_EDGES = 320000
D = 16
K_PROP = 10
ALPHA = 0.1

NS = 16
ET = N_EDGES // NS
CHUNK = 1000
NCHUNK = ET // CHUNK
N_PAD = 10240
STRIPE = N_PAD // NS

_MESH = plsc.VectorSubcoreMesh(
    core_axis_name="c", subcore_axis_name="s", num_cores=1
)
_SC_PARAMS = pltpu.CompilerParams(use_tc_tiling_on_sc=False)


def _deg_body(dst_hbm, deg_hbm, dst_v, ones_v, s_sp):
    t = lax.axis_index("s")
    stripe = pl.ds(t * STRIPE, STRIPE)

    pltpu.sync_copy(dst_hbm.at[t], dst_v)

    def fill_one(i, _):
        ones_v[i, :] = jnp.full((D,), 1.0, jnp.float32)
        return ()
    lax.fori_loop(0, CHUNK, fill_one, ())

    pltpu.sync_copy(ones_v.at[pl.ds(0, STRIPE)], s_sp.at[stripe])
    plsc.subcore_barrier()

    def deg_chunk(j, _):
        pltpu.sync_copy(ones_v, s_sp.at[dst_v.at[j]], add=True)
        return ()
    lax.fori_loop(0, NCHUNK, deg_chunk, ())
    plsc.subcore_barrier()

    pltpu.sync_copy(s_sp.at[stripe], deg_hbm.at[stripe])


def _tc_body(x_ref, w1_ref, b1_ref, w2_ref, b2_ref, deg_ref,
             g0_ref, c1_ref, c2_ref, c3_ref, c4_ref):
    h = jnp.dot(x_ref[...], w1_ref[...], preferred_element_type=jnp.float32)
    h = jnp.maximum(h + b1_ref[...], 0.0)
    h0 = jnp.dot(h, w2_ref[...], preferred_element_type=jnp.float32) + b2_ref[...]
    dis = lax.rsqrt(deg_ref[...])
    g0_ref[...] = dis * h0
    c1_ref[...] = (1.0 - ALPHA) * dis * dis
    c2_ref[...] = ALPHA * dis * h0
    c3_ref[...] = (1.0 - ALPHA) * dis
    c4_ref[...] = ALPHA * h0


def _tc_stage(x, W1, b1, W2, b2, deg_b):
    n, d_in = x.shape
    hid = W1.shape[1]
    blk = 1024
    full = lambda shape: pl.BlockSpec(shape, lambda i: (0, 0))
    row = pl.BlockSpec((blk, D), lambda i: (i, 0))
    out_sds = jax.ShapeDtypeStruct((n, D), jnp.float32)
    return pl.pallas_call(
        _tc_body,
        grid=(n // blk,),
        in_specs=[
            pl.BlockSpec((blk, d_in), lambda i: (i, 0)),
            full((d_in, hid)),
            full((1, hid)),
            full((hid, D)),
            full((1, D)),
            row,
        ],
        out_specs=(row, row, row, row, row),
        out_shape=(out_sds,) * 5,
    )(x, W1, b1.reshape(1, hid), W2, b2.reshape(1, D), deg_b)


def _appnp_body(g0_hbm, c1_hbm, c2_hbm, c3_hbm, c4_hbm, src_hbm, dst_hbm,
                out_hbm, g_hbm,
                src_v, dst_v, rows_a, rows_b, c1_v, c2_v,
                s_sp, sem_ga, sem_gb, sem_sa, sem_sb):
    t = lax.axis_index("s")
    stripe = pl.ds(t * STRIPE, STRIPE)
    rS = pl.ds(0, STRIPE)
    bufs = (rows_a, rows_b)
    gsems = (sem_ga, sem_gb)
    ssems = (sem_sa, sem_sb)

    pltpu.sync_copy(src_hbm.at[t], src_v)
    pltpu.sync_copy(dst_hbm.at[t], dst_v)
    pltpu.sync_copy(c1_hbm.at[stripe], c1_v)
    pltpu.sync_copy(c2_hbm.at[stripe], c2_v)

    pltpu.sync_copy(g0_hbm.at[stripe], rows_a.at[rS])
    pltpu.sync_copy(rows_a.at[rS], g_hbm.at[stripe])
    pltpu.sync_copy(rows_a.at[rS], s_sp.at[stripe])
    plsc.subcore_barrier()

    def start_gather(j, b):
        pltpu.async_copy(g_hbm.at[src_v.at[j]], bufs[b], gsems[b])

    def wait_gather(b):
        pltpu.make_async_copy(g_hbm.at[src_v.at[0]], bufs[b], gsems[b]).wait()

    def start_scatter(j, b):
        pltpu.async_copy(bufs[b], s_sp.at[dst_v.at[j]], ssems[b], add=True)

    def wait_scatter(b):
        pltpu.make_async_copy(bufs[b], s_sp.at[dst_v.at[0]], ssems[b]).wait()

    def edge_pipe(jj, _):
        j = jj * 2
        wait_gather(0)
        start_scatter(j, 0)
        start_gather(j + 1, 1)
        wait_gather(1)
        start_scatter(j + 1, 1)
        wait_scatter(0)

        @pl.when(jj < NCHUNK // 2 - 1)
        def _():
            start_gather(j + 2, 0)
        wait_scatter(1)
        return ()

    def update(i, _):
        rows_a[i, :] = c1_v[i, :] * rows_a[i, :] + c2_v[i, :]
        return ()

    for k in range(K_PROP):
        start_gather(0, 0)
        lax.fori_loop(0, NCHUNK // 2, edge_pipe, ())
        plsc.subcore_barrier()

        pltpu.sync_copy(s_sp.at[stripe], rows_a.at[rS])
        if k < K_PROP - 1:
            lax.fori_loop(0, STRIPE, update, (), unroll=8)
            pltpu.sync_copy(rows_a.at[rS], g_hbm.at[stripe])
            pltpu.sync_copy(rows_a.at[rS], s_sp.at[stripe])
            plsc.subcore_barrier()
        else:
            pltpu.sync_copy(c3_hbm.at[stripe], c1_v)
            pltpu.sync_copy(c4_hbm.at[stripe], c2_v)
            lax.fori_loop(0, STRIPE, update, (), unroll=8)
            pltpu.sync_copy(rows_a.at[rS], out_hbm.at[stripe])


@jax.jit
def _run(x, src, dst, W1, b1, W2, b2):
    deg_kernel = pl.kernel(
        _deg_body,
        out_type=jax.ShapeDtypeStruct((N_PAD, D), jnp.float32),
        mesh=_MESH,
        compiler_params=_SC_PARAMS,
        scratch_types=[
            pltpu.VMEM((NCHUNK, CHUNK), jnp.int32),
            pltpu.VMEM((CHUNK, D), jnp.float32),
            pltpu.VMEM_SHARED((N_PAD, D), jnp.float32),
        ],
    )
    deg_b = deg_kernel(dst)

    xp = jnp.concatenate(
        [x, jnp.zeros((N_PAD - N_NODES, x.shape[1]), x.dtype)], axis=0
    )
    g0, c1, c2, c3, c4 = _tc_stage(xp, W1, b1, W2, b2, deg_b)

    appnp = pl.kernel(
        _appnp_body,
        out_type=(
            jax.ShapeDtypeStruct((N_PAD, D), jnp.float32),
            jax.ShapeDtypeStruct((N_PAD, D), jnp.float32),
        ),
        mesh=_MESH,
        compiler_params=_SC_PARAMS,
        scratch_types=[
            pltpu.VMEM((NCHUNK, CHUNK), jnp.int32),
            pltpu.VMEM((NCHUNK, CHUNK), jnp.int32),
            pltpu.VMEM((CHUNK, D), jnp.float32),
            pltpu.VMEM((CHUNK, D), jnp.float32),
            pltpu.VMEM((STRIPE, D), jnp.float32),
            pltpu.VMEM((STRIPE, D), jnp.float32),
            pltpu.VMEM_SHARED((N_PAD, D), jnp.float32),
            pltpu.SemaphoreType.DMA,
            pltpu.SemaphoreType.DMA,
            pltpu.SemaphoreType.DMA,
            pltpu.SemaphoreType.DMA,
        ],
    )
    out, _ = appnp(g0, c1, c2, c3, c4, src, dst)
    return out[:N_NODES]


def kernel(x, edge_index, epoch, W1, b1, W2, b2):
    src = edge_index[0].astype(jnp.int32).reshape(NS, NCHUNK, CHUNK)
    dst = edge_index[1].astype(jnp.int32).reshape(NS, NCHUNK, CHUNK)
    return _run(x, src, dst, W1, b1, W2, b2)


# speedup vs baseline: 57.9935x; 1.2030x over previous
import jax
import jax.numpy as jnp
from jax import lax
from jax.experimental import pallas as pl
from jax.experimental.pallas import tpu as pltpu
from jax.experimental.pallas import tpu_sc as plsc

N_NODES = 10000
N_EDGES = 320000
D = 16
K_PROP = 10
ALPHA = 0.1

NS = 16
ET = N_EDGES // NS
CHUNK = 1000
NCHUNK = ET // CHUNK
N_PAD = 10240
STRIPE = N_PAD // NS

_MESH = plsc.VectorSubcoreMesh(
    core_axis_name="c", subcore_axis_name="s", num_cores=1
)
_SC_PARAMS = pltpu.CompilerParams(use_tc_tiling_on_sc=False)


def _deg_body(dst_hbm, deg_hbm, dst_v, ones_v, s_sp):
    t = lax.axis_index("s")
    stripe = pl.ds(t * STRIPE, STRIPE)

    pltpu.sync_copy(dst_hbm.at[t], dst_v)

    def fill_one(i, _):
        ones_v[i, :] = jnp.full((D,), 1.0, jnp.float32)
        return ()
    lax.fori_loop(0, CHUNK, fill_one, ())

    pltpu.sync_copy(ones_v.at[pl.ds(0, STRIPE)], s_sp.at[stripe])
    plsc.subcore_barrier()

    def deg_chunk(j, _):
        pltpu.sync_copy(ones_v, s_sp.at[dst_v.at[j]], add=True)
        return ()
    lax.fori_loop(0, NCHUNK, deg_chunk, ())
    plsc.subcore_barrier()

    pltpu.sync_copy(s_sp.at[stripe], deg_hbm.at[stripe])


def _tc_body(x_ref, w1_ref, b1_ref, w2_ref, b2_ref, deg_ref,
             g0_ref, c1_ref, c2_ref, c3_ref, c4_ref):
    h = jnp.dot(x_ref[...], w1_ref[...], preferred_element_type=jnp.float32)
    h = jnp.maximum(h + b1_ref[...], 0.0)
    h0 = jnp.dot(h, w2_ref[...], preferred_element_type=jnp.float32) + b2_ref[...]
    dis = lax.rsqrt(deg_ref[...])
    g0_ref[...] = dis * h0
    c1_ref[...] = (1.0 - ALPHA) * dis * dis
    c2_ref[...] = ALPHA * dis * h0
    c3_ref[...] = (1.0 - ALPHA) * dis
    c4_ref[...] = ALPHA * h0


def _tc_stage(x, W1, b1, W2, b2, deg_b):
    n, d_in = x.shape
    hid = W1.shape[1]
    blk = 1024
    full = lambda shape: pl.BlockSpec(shape, lambda i: (0, 0))
    row = pl.BlockSpec((blk, D), lambda i: (i, 0))
    out_sds = jax.ShapeDtypeStruct((n, D), jnp.float32)
    return pl.pallas_call(
        _tc_body,
        grid=(n // blk,),
        in_specs=[
            pl.BlockSpec((blk, d_in), lambda i: (i, 0)),
            full((d_in, hid)),
            full((1, hid)),
            full((hid, D)),
            full((1, D)),
            row,
        ],
        out_specs=(row, row, row, row, row),
        out_shape=(out_sds,) * 5,
    )(x, W1, b1.reshape(1, hid), W2, b2.reshape(1, D), deg_b)


def _appnp_body(g0_hbm, c1_hbm, c2_hbm, c3_hbm, c4_hbm, src_hbm, dst_hbm,
                out_hbm,
                src_v, dst_v, rows_a, rows_b, c1_v, c2_v,
                s_sp, g_sp, sem_ga, sem_gb, sem_sa, sem_sb):
    t = lax.axis_index("s")
    stripe = pl.ds(t * STRIPE, STRIPE)
    rS = pl.ds(0, STRIPE)
    bufs = (rows_a, rows_b)
    gsems = (sem_ga, sem_gb)
    ssems = (sem_sa, sem_sb)

    pltpu.sync_copy(src_hbm.at[t], src_v)
    pltpu.sync_copy(dst_hbm.at[t], dst_v)
    pltpu.sync_copy(c1_hbm.at[stripe], c1_v)
    pltpu.sync_copy(c2_hbm.at[stripe], c2_v)

    pltpu.sync_copy(g0_hbm.at[stripe], rows_a.at[rS])
    pltpu.sync_copy(rows_a.at[rS], g_sp.at[stripe])
    pltpu.sync_copy(rows_a.at[rS], s_sp.at[stripe])
    plsc.subcore_barrier()

    def start_gather(j, b):
        pltpu.async_copy(g_sp.at[src_v.at[j]], bufs[b], gsems[b])

    def wait_gather(b):
        pltpu.make_async_copy(g_sp.at[src_v.at[0]], bufs[b], gsems[b]).wait()

    def start_scatter(j, b):
        pltpu.async_copy(bufs[b], s_sp.at[dst_v.at[j]], ssems[b], add=True)

    def wait_scatter(b):
        pltpu.make_async_copy(bufs[b], s_sp.at[dst_v.at[0]], ssems[b]).wait()

    def edge_pipe(jj, _):
        j = jj * 2
        wait_gather(0)
        start_scatter(j, 0)
        start_gather(j + 1, 1)
        wait_gather(1)
        start_scatter(j + 1, 1)
        wait_scatter(0)

        @pl.when(jj < NCHUNK // 2 - 1)
        def _():
            start_gather(j + 2, 0)
        wait_scatter(1)
        return ()

    def update(i, _):
        rows_a[i, :] = c1_v[i, :] * rows_a[i, :] + c2_v[i, :]
        return ()

    for k in range(K_PROP):
        start_gather(0, 0)
        lax.fori_loop(0, NCHUNK // 2, edge_pipe, ())
        plsc.subcore_barrier()

        pltpu.sync_copy(s_sp.at[stripe], rows_a.at[rS])
        if k < K_PROP - 1:
            lax.fori_loop(0, STRIPE, update, (), unroll=8)
            pltpu.sync_copy(rows_a.at[rS], g_sp.at[stripe])
            pltpu.sync_copy(rows_a.at[rS], s_sp.at[stripe])
            plsc.subcore_barrier()
        else:
            pltpu.sync_copy(c3_hbm.at[stripe], c1_v)
            pltpu.sync_copy(c4_hbm.at[stripe], c2_v)
            lax.fori_loop(0, STRIPE, update, (), unroll=8)
            pltpu.sync_copy(rows_a.at[rS], out_hbm.at[stripe])


@jax.jit
def _run(x, src, dst, W1, b1, W2, b2):
    deg_kernel = pl.kernel(
        _deg_body,
        out_type=jax.ShapeDtypeStruct((N_PAD, D), jnp.float32),
        mesh=_MESH,
        compiler_params=_SC_PARAMS,
        scratch_types=[
            pltpu.VMEM((NCHUNK, CHUNK), jnp.int32),
            pltpu.VMEM((CHUNK, D), jnp.float32),
            pltpu.VMEM_SHARED((N_PAD, D), jnp.float32),
        ],
    )
    deg_b = deg_kernel(dst)

    xp = jnp.concatenate(
        [x, jnp.zeros((N_PAD - N_NODES, x.shape[1]), x.dtype)], axis=0
    )
    g0, c1, c2, c3, c4 = _tc_stage(xp, W1, b1, W2, b2, deg_b)

    appnp = pl.kernel(
        _appnp_body,
        out_type=jax.ShapeDtypeStruct((N_PAD, D), jnp.float32),
        mesh=_MESH,
        compiler_params=_SC_PARAMS,
        scratch_types=[
            pltpu.VMEM((NCHUNK, CHUNK), jnp.int32),
            pltpu.VMEM((NCHUNK, CHUNK), jnp.int32),
            pltpu.VMEM((CHUNK, D), jnp.float32),
            pltpu.VMEM((CHUNK, D), jnp.float32),
            pltpu.VMEM((STRIPE, D), jnp.float32),
            pltpu.VMEM((STRIPE, D), jnp.float32),
            pltpu.VMEM_SHARED((N_PAD, D), jnp.float32),
            pltpu.VMEM_SHARED((N_PAD, D), jnp.float32),
            pltpu.SemaphoreType.DMA,
            pltpu.SemaphoreType.DMA,
            pltpu.SemaphoreType.DMA,
            pltpu.SemaphoreType.DMA,
        ],
    )
    out = appnp(g0, c1, c2, c3, c4, src, dst)
    return out[:N_NODES]


def kernel(x, edge_index, epoch, W1, b1, W2, b2):
    src = edge_index[0].astype(jnp.int32).reshape(NS, NCHUNK, CHUNK)
    dst = edge_index[1].astype(jnp.int32).reshape(NS, NCHUNK, CHUNK)
    return _run(x, src, dst, W1, b1, W2, b2)


# speedup vs baseline: 58.3012x; 1.0053x over previous
import jax
import jax.numpy as jnp
from jax import lax
from jax.experimental import pallas as pl
from jax.experimental.pallas import tpu as pltpu
from jax.experimental.pallas import tpu_sc as plsc

N_NODES = 10000
N_EDGES = 320000
D = 16
K_PROP = 10
ALPHA = 0.1

NS = 16
ET = N_EDGES // NS
CHUNK = 1000
NCHUNK = ET // CHUNK
N_PAD = 10240
STRIPE = N_PAD // NS

_MESH = plsc.VectorSubcoreMesh(
    core_axis_name="c", subcore_axis_name="s", num_cores=1
)
_SC_PARAMS = pltpu.CompilerParams(use_tc_tiling_on_sc=False)


def _mlp_body(x_ref, w1_ref, b1_ref, w2_ref, b2_ref, o_ref):
    h = jnp.dot(x_ref[...], w1_ref[...], preferred_element_type=jnp.float32)
    h = jnp.maximum(h + b1_ref[...], 0.0)
    o_ref[...] = (
        jnp.dot(h, w2_ref[...], preferred_element_type=jnp.float32) + b2_ref[...]
    )


def _mlp(x, W1, b1, W2, b2):
    n, d_in = x.shape
    hid = W1.shape[1]
    blk = 1024
    full = lambda shape: pl.BlockSpec(shape, lambda i: (0, 0))
    return pl.pallas_call(
        _mlp_body,
        grid=(n // blk,),
        in_specs=[
            pl.BlockSpec((blk, d_in), lambda i: (i, 0)),
            full((d_in, hid)),
            full((1, hid)),
            full((hid, D)),
            full((1, D)),
        ],
        out_specs=pl.BlockSpec((blk, D), lambda i: (i, 0)),
        out_shape=jax.ShapeDtypeStruct((n, D), jnp.float32),
    )(x, W1, b1.reshape(1, hid), W2, b2.reshape(1, D))


def _rsqrt_rows(m):
    s = jnp.full((D,), 1.0, jnp.float32)
    for f, r in ((65536.0, 1.0 / 256), (256.0, 1.0 / 16), (16.0, 0.25),
                 (4.0, 0.5)):
        c = m >= f
        m = jnp.where(c, m * (1.0 / f), m)
        s = jnp.where(c, s * r, s)
    y = 1.074 - 0.18 * m
    for _ in range(4):
        y = y * (1.5 - 0.5 * m * y * y)
    return y * s


def _appnp_body(h0_hbm, src_hbm, dst_hbm, out_hbm,
                src_v, dst_v, rows_a, rows_b, c1_v, dis_v, c4_v,
                s_sp, g_sp, sem_ga, sem_gb, sem_sa, sem_sb):
    t = lax.axis_index("s")
    stripe = pl.ds(t * STRIPE, STRIPE)
    rS = pl.ds(0, STRIPE)
    bufs = (rows_a, rows_b)
    gsems = (sem_ga, sem_gb)
    ssems = (sem_sa, sem_sb)

    pltpu.sync_copy(src_hbm.at[t], src_v)
    pltpu.sync_copy(dst_hbm.at[t], dst_v)
    pltpu.sync_copy(h0_hbm.at[stripe], c4_v)

    def fill_one(i, _):
        rows_a[i, :] = jnp.full((D,), 1.0, jnp.float32)
        return ()
    lax.fori_loop(0, CHUNK, fill_one, ())

    pltpu.sync_copy(rows_a.at[rS], s_sp.at[stripe])
    plsc.subcore_barrier()

    def deg_chunk(j, _):
        pltpu.sync_copy(rows_a, s_sp.at[dst_v.at[j]], add=True)
        return ()
    lax.fori_loop(0, NCHUNK, deg_chunk, ())
    plsc.subcore_barrier()

    pltpu.sync_copy(s_sp.at[stripe], rows_a.at[rS])

    def coeffs(i, _):
        y = _rsqrt_rows(rows_a[i, :])
        h0r = c4_v[i, :]
        dis_v[i, :] = y
        c1_v[i, :] = (1.0 - ALPHA) * y * y
        c4_v[i, :] = ALPHA * h0r
        rows_a[i, :] = y * h0r
        return ()
    lax.fori_loop(0, STRIPE, coeffs, (), unroll=4)

    pltpu.sync_copy(rows_a.at[rS], g_sp.at[stripe])
    pltpu.sync_copy(rows_a.at[rS], s_sp.at[stripe])
    plsc.subcore_barrier()

    def start_gather(j, b):
        pltpu.async_copy(g_sp.at[src_v.at[j]], bufs[b], gsems[b])

    def wait_gather(b):
        pltpu.make_async_copy(g_sp.at[src_v.at[0]], bufs[b], gsems[b]).wait()

    def start_scatter(j, b):
        pltpu.async_copy(bufs[b], s_sp.at[dst_v.at[j]], ssems[b], add=True)

    def wait_scatter(b):
        pltpu.make_async_copy(bufs[b], s_sp.at[dst_v.at[0]], ssems[b]).wait()

    def edge_pipe(jj, _):
        j = jj * 2
        wait_gather(0)
        start_scatter(j, 0)
        start_gather(j + 1, 1)
        wait_gather(1)
        start_scatter(j + 1, 1)
        wait_scatter(0)

        @pl.when(jj < NCHUNK // 2 - 1)
        def _():
            start_gather(j + 2, 0)
        wait_scatter(1)
        return ()

    def update(i, _):
        rows_a[i, :] = (c1_v[i, :] * rows_a[i, :]
                        + dis_v[i, :] * c4_v[i, :])
        return ()

    def final(i, _):
        rows_a[i, :] = ((1.0 - ALPHA) * dis_v[i, :] * rows_a[i, :]
                        + c4_v[i, :])
        return ()

    for k in range(K_PROP):
        start_gather(0, 0)
        lax.fori_loop(0, NCHUNK // 2, edge_pipe, ())
        plsc.subcore_barrier()

        pltpu.sync_copy(s_sp.at[stripe], rows_a.at[rS])
        if k < K_PROP - 1:
            lax.fori_loop(0, STRIPE, update, (), unroll=8)
            pltpu.sync_copy(rows_a.at[rS], g_sp.at[stripe])
            pltpu.sync_copy(rows_a.at[rS], s_sp.at[stripe])
            plsc.subcore_barrier()
        else:
            lax.fori_loop(0, STRIPE, final, (), unroll=8)
            pltpu.sync_copy(rows_a.at[rS], out_hbm.at[stripe])


@jax.jit
def _run(x, src, dst, W1, b1, W2, b2):
    xp = jnp.concatenate(
        [x, jnp.zeros((N_PAD - N_NODES, x.shape[1]), x.dtype)], axis=0
    )
    h0 = _mlp(xp, W1, b1, W2, b2)

    appnp = pl.kernel(
        _appnp_body,
        out_type=jax.ShapeDtypeStruct((N_PAD, D), jnp.float32),
        mesh=_MESH,
        compiler_params=_SC_PARAMS,
        scratch_types=[
            pltpu.VMEM((NCHUNK, CHUNK), jnp.int32),
            pltpu.VMEM((NCHUNK, CHUNK), jnp.int32),
            pltpu.VMEM((CHUNK, D), jnp.float32),
            pltpu.VMEM((CHUNK, D), jnp.float32),
            pltpu.VMEM((STRIPE, D), jnp.float32),
            pltpu.VMEM((STRIPE, D), jnp.float32),
            pltpu.VMEM((STRIPE, D), jnp.float32),
            pltpu.VMEM_SHARED((N_PAD, D), jnp.float32),
            pltpu.VMEM_SHARED((N_PAD, D), jnp.float32),
            pltpu.SemaphoreType.DMA,
            pltpu.SemaphoreType.DMA,
            pltpu.SemaphoreType.DMA,
            pltpu.SemaphoreType.DMA,
        ],
    )
    out = appnp(h0, src, dst)
    return out[:N_NODES]


def kernel(x, edge_index, epoch, W1, b1, W2, b2):
    src = edge_index[0].astype(jnp.int32).reshape(NS, NCHUNK, CHUNK)
    dst = edge_index[1].astype(jnp.int32).reshape(NS, NCHUNK, CHUNK)
    return _run(x, src, dst, W1, b1, W2, b2)


# speedup vs baseline: 77.5445x; 1.3301x over previous
import jax
import jax.numpy as jnp
from jax import lax
from jax.experimental import pallas as pl
from jax.experimental.pallas import tpu as pltpu
from jax.experimental.pallas import tpu_sc as plsc

N_NODES = 10000
N_EDGES = 320000
D = 16
K_PROP = 10
ALPHA = 0.1

NC = 2
NS = 16
NW = NC * NS
ET = N_EDGES // NW
CHUNK = 1000
NCHUNK = ET // CHUNK
N_PAD = 10240
STRIPE = N_PAD // NS
NSLOT = K_PROP + 1

_MESH = plsc.VectorSubcoreMesh(
    core_axis_name="c", subcore_axis_name="s", num_cores=NC
)
_SC_PARAMS = pltpu.CompilerParams(use_tc_tiling_on_sc=False)


def _mlp_body(x_ref, w1_ref, b1_ref, w2_ref, b2_ref, o_ref):
    h = jnp.dot(x_ref[...], w1_ref[...], preferred_element_type=jnp.float32)
    h = jnp.maximum(h + b1_ref[...], 0.0)
    o_ref[...] = (
        jnp.dot(h, w2_ref[...], preferred_element_type=jnp.float32) + b2_ref[...]
    )


def _mlp(x, W1, b1, W2, b2):
    n, d_in = x.shape
    hid = W1.shape[1]
    blk = 1024
    full = lambda shape: pl.BlockSpec(shape, lambda i: (0, 0))
    return pl.pallas_call(
        _mlp_body,
        grid=(n // blk,),
        in_specs=[
            pl.BlockSpec((blk, d_in), lambda i: (i, 0)),
            full((d_in, hid)),
            full((1, hid)),
            full((hid, D)),
            full((1, D)),
        ],
        out_specs=pl.BlockSpec((blk, D), lambda i: (i, 0)),
        out_shape=jax.ShapeDtypeStruct((n, D), jnp.float32),
    )(x, W1, b1.reshape(1, hid), W2, b2.reshape(1, D))


def _rsqrt_rows(m):
    s = jnp.full((D,), 1.0, jnp.float32)
    for f, r in ((65536.0, 1.0 / 256), (256.0, 1.0 / 16), (16.0, 0.25),
                 (4.0, 0.5)):
        c = m >= f
        m = jnp.where(c, m * (1.0 / f), m)
        s = jnp.where(c, s * r, s)
    y = 1.074 - 0.18 * m
    for _ in range(4):
        y = y * (1.5 - 0.5 * m * y * y)
    return y * s


def _appnp_body(h0_hbm, src_hbm, dst_hbm, out_hbm, x_hbm,
                src_v, dst_v, rows_a, rows_b, zero_v, c1_v, dis_v, c4_v,
                s_sp, g_sp, sem_ga, sem_gb, sem_sa, sem_sb, xsem):
    c = lax.axis_index("c")
    t = lax.axis_index("s")
    w = c * NS + t
    stripe = pl.ds(t * STRIPE, STRIPE)
    rS = pl.ds(0, STRIPE)
    bufs = (rows_a, rows_b)
    gsems = (sem_ga, sem_gb)
    ssems = (sem_sa, sem_sb)

    pltpu.sync_copy(src_hbm.at[w], src_v)
    pltpu.sync_copy(dst_hbm.at[w], dst_v)
    pltpu.sync_copy(h0_hbm.at[stripe], c4_v)

    def fill_one(i, _):
        rows_a[i, :] = jnp.full((D,), 1.0, jnp.float32)
        return ()
    lax.fori_loop(0, CHUNK, fill_one, ())

    def fill_zero(i, _):
        zero_v[i, :] = jnp.full((D,), 0.0, jnp.float32)
        return ()
    lax.fori_loop(0, STRIPE, fill_zero, ())

    def exchange(slot):
        p = slot % 2
        pltpu.sync_copy(s_sp.at[stripe], x_hbm.at[c, p, stripe])
        plsc.subcore_barrier()
        pltpu.semaphore_signal(xsem, 1, core_index=1 - c)
        pltpu.semaphore_wait(xsem, 1)

        pltpu.sync_copy(s_sp.at[stripe], rows_a.at[rS])
        pltpu.sync_copy(x_hbm.at[1 - c, p, stripe], rows_b.at[rS])

    @pl.when(c == 0)
    def _():
        pltpu.sync_copy(rows_a.at[rS], s_sp.at[stripe])

    @pl.when(c != 0)
    def _():
        pltpu.sync_copy(zero_v, s_sp.at[stripe])
    plsc.subcore_barrier()

    def deg_chunk(j, _):
        pltpu.sync_copy(rows_a, s_sp.at[dst_v.at[j]], add=True)
        return ()
    lax.fori_loop(0, NCHUNK, deg_chunk, ())
    plsc.subcore_barrier()

    exchange(0)

    def coeffs(i, _):
        y = _rsqrt_rows(rows_a[i, :] + rows_b[i, :])
        h0r = c4_v[i, :]
        dis_v[i, :] = y
        c1_v[i, :] = (1.0 - ALPHA) * y * y
        c4_v[i, :] = ALPHA * h0r
        rows_a[i, :] = y * h0r
        return ()
    lax.fori_loop(0, STRIPE, coeffs, (), unroll=4)

    def seed_round(k_is_last):
        del k_is_last
        pltpu.sync_copy(rows_a.at[rS], g_sp.at[stripe])

        @pl.when(c == 0)
        def _():
            pltpu.sync_copy(rows_a.at[rS], s_sp.at[stripe])

        @pl.when(c != 0)
        def _():
            pltpu.sync_copy(zero_v, s_sp.at[stripe])
        plsc.subcore_barrier()

    seed_round(False)

    def start_gather(j, b):
        pltpu.async_copy(g_sp.at[src_v.at[j]], bufs[b], gsems[b])

    def wait_gather(b):
        pltpu.make_async_copy(g_sp.at[src_v.at[0]], bufs[b], gsems[b]).wait()

    def start_scatter(j, b):
        pltpu.async_copy(bufs[b], s_sp.at[dst_v.at[j]], ssems[b], add=True)

    def wait_scatter(b):
        pltpu.make_async_copy(bufs[b], s_sp.at[dst_v.at[0]], ssems[b]).wait()

    def edge_pipe(jj, _):
        j = jj * 2
        wait_gather(0)
        start_scatter(j, 0)
        start_gather(j + 1, 1)
        wait_gather(1)
        start_scatter(j + 1, 1)
        wait_scatter(0)

        @pl.when(jj < NCHUNK // 2 - 1)
        def _():
            start_gather(j + 2, 0)
        wait_scatter(1)
        return ()

    def update(i, _):
        rows_a[i, :] = (c1_v[i, :] * (rows_a[i, :] + rows_b[i, :])
                        + dis_v[i, :] * c4_v[i, :])
        return ()

    def final(i, _):
        rows_a[i, :] = ((1.0 - ALPHA) * dis_v[i, :]
                        * (rows_a[i, :] + rows_b[i, :]) + c4_v[i, :])
        return ()

    for k in range(K_PROP):
        start_gather(0, 0)
        lax.fori_loop(0, NCHUNK // 2, edge_pipe, ())
        plsc.subcore_barrier()

        exchange(k + 1)
        if k < K_PROP - 1:
            lax.fori_loop(0, STRIPE, update, (), unroll=8)
            seed_round(False)
        else:
            lax.fori_loop(0, STRIPE, final, (), unroll=8)

            @pl.when(c == 0)
            def _():
                pltpu.sync_copy(rows_a.at[rS], out_hbm.at[stripe])


@jax.jit
def _run(x, src, dst, W1, b1, W2, b2):
    xp = jnp.concatenate(
        [x, jnp.zeros((N_PAD - N_NODES, x.shape[1]), x.dtype)], axis=0
    )
    h0 = _mlp(xp, W1, b1, W2, b2)

    appnp = pl.kernel(
        _appnp_body,
        out_type=(
            jax.ShapeDtypeStruct((N_PAD, D), jnp.float32),
            jax.ShapeDtypeStruct((NC, 2, N_PAD, D), jnp.float32),
        ),
        mesh=_MESH,
        compiler_params=_SC_PARAMS,
        scratch_types=[
            pltpu.VMEM((NCHUNK, CHUNK), jnp.int32),
            pltpu.VMEM((NCHUNK, CHUNK), jnp.int32),
            pltpu.VMEM((CHUNK, D), jnp.float32),
            pltpu.VMEM((CHUNK, D), jnp.float32),
            pltpu.VMEM((STRIPE, D), jnp.float32),
            pltpu.VMEM((STRIPE, D), jnp.float32),
            pltpu.VMEM((STRIPE, D), jnp.float32),
            pltpu.VMEM((STRIPE, D), jnp.float32),
            pltpu.VMEM_SHARED((N_PAD, D), jnp.float32),
            pltpu.VMEM_SHARED((N_PAD, D), jnp.float32),
            pltpu.SemaphoreType.DMA,
            pltpu.SemaphoreType.DMA,
            pltpu.SemaphoreType.DMA,
            pltpu.SemaphoreType.DMA,
            pltpu.SemaphoreType.REGULAR,
        ],
    )
    out, _ = appnp(h0, src, dst)
    return out[:N_NODES]


def kernel(x, edge_index, epoch, W1, b1, W2, b2):
    src = edge_index[0].astype(jnp.int32).reshape(NW, NCHUNK, CHUNK)
    dst = edge_index[1].astype(jnp.int32).reshape(NW, NCHUNK, CHUNK)
    return _run(x, src, dst, W1, b1, W2, b2)


# speedup vs baseline: 79.2740x; 1.0223x over previous
import jax
import jax.numpy as jnp
from jax import lax
from jax.experimental import pallas as pl
from jax.experimental.pallas import tpu as pltpu
from jax.experimental.pallas import tpu_sc as plsc

N_NODES = 10000
N_EDGES = 320000
D = 16
K_PROP = 10
ALPHA = 0.1

NC = 2
NS = 16
NW = NC * NS
ET = N_EDGES // NW
CHUNK = 1000
NCHUNK = ET // CHUNK
N_PAD = 10240
STRIPE = N_PAD // NS
LAST_H = N_NODES - (NS - 1) * STRIPE
NSLOT = K_PROP + 1

_MESH = plsc.VectorSubcoreMesh(
    core_axis_name="c", subcore_axis_name="s", num_cores=NC
)
_SC_PARAMS = pltpu.CompilerParams(use_tc_tiling_on_sc=False)


def _mlp_body(x_ref, w1_ref, b1_ref, w2_ref, b2_ref, o_ref):
    h = jnp.dot(x_ref[...], w1_ref[...], preferred_element_type=jnp.float32)
    h = jnp.maximum(h + b1_ref[...], 0.0)
    o_ref[...] = (
        jnp.dot(h, w2_ref[...], preferred_element_type=jnp.float32) + b2_ref[...]
    )


def _mlp(x, W1, b1, W2, b2):
    n, d_in = x.shape
    hid = W1.shape[1]
    blk = 1000
    full = lambda shape: pl.BlockSpec(shape, lambda i: (0, 0))
    return pl.pallas_call(
        _mlp_body,
        grid=(n // blk,),
        in_specs=[
            pl.BlockSpec((blk, d_in), lambda i: (i, 0)),
            full((d_in, hid)),
            full((1, hid)),
            full((hid, D)),
            full((1, D)),
        ],
        out_specs=pl.BlockSpec((blk, D), lambda i: (i, 0)),
        out_shape=jax.ShapeDtypeStruct((n, D), jnp.float32),
    )(x, W1, b1.reshape(1, hid), W2, b2.reshape(1, D))


def _rsqrt_rows(m):
    s = jnp.full((D,), 1.0, jnp.float32)
    for f, r in ((65536.0, 1.0 / 256), (256.0, 1.0 / 16), (16.0, 0.25),
                 (4.0, 0.5)):
        c = m >= f
        m = jnp.where(c, m * (1.0 / f), m)
        s = jnp.where(c, s * r, s)
    y = 1.074 - 0.18 * m
    for _ in range(4):
        y = y * (1.5 - 0.5 * m * y * y)
    return y * s


def _appnp_body(h0_hbm, src_hbm, dst_hbm, out_hbm, x_hbm,
                src_v, dst_v, rows_a, rows_b, zero_v, c1_v, dis_v, c4_v,
                s_sp, g_sp, sem_ga, sem_gb, sem_sa, sem_sb, xsem):
    c = lax.axis_index("c")
    t = lax.axis_index("s")
    w = c * NS + t
    stripe = pl.ds(t * STRIPE, STRIPE)
    rS = pl.ds(0, STRIPE)
    bufs = (rows_a, rows_b)
    gsems = (sem_ga, sem_gb)
    ssems = (sem_sa, sem_sb)

    pltpu.sync_copy(src_hbm.at[w], src_v)
    pltpu.sync_copy(dst_hbm.at[w], dst_v)

    @pl.when(t < NS - 1)
    def _():
        pltpu.sync_copy(h0_hbm.at[stripe], c4_v)

    @pl.when(t == NS - 1)
    def _():
        pltpu.sync_copy(h0_hbm.at[pl.ds((NS - 1) * STRIPE, LAST_H)],
                        c4_v.at[pl.ds(0, LAST_H)])

    def fill_one(i, _):
        rows_a[i, :] = jnp.full((D,), 1.0, jnp.float32)
        return ()
    lax.fori_loop(0, CHUNK, fill_one, ())

    def fill_zero(i, _):
        zero_v[i, :] = jnp.full((D,), 0.0, jnp.float32)
        return ()
    lax.fori_loop(0, STRIPE, fill_zero, ())

    def exchange(slot):
        p = slot % 2
        pltpu.sync_copy(s_sp.at[stripe], x_hbm.at[c, p, stripe])
        plsc.subcore_barrier()
        pltpu.semaphore_signal(xsem, 1, core_index=1 - c)
        pltpu.semaphore_wait(xsem, 1)

        pltpu.sync_copy(s_sp.at[stripe], rows_a.at[rS])
        pltpu.sync_copy(x_hbm.at[1 - c, p, stripe], rows_b.at[rS])

    @pl.when(c == 0)
    def _():
        pltpu.sync_copy(rows_a.at[rS], s_sp.at[stripe])

    @pl.when(c != 0)
    def _():
        pltpu.sync_copy(zero_v, s_sp.at[stripe])
    plsc.subcore_barrier()

    def deg_chunk(j, _):
        pltpu.sync_copy(rows_a, s_sp.at[dst_v.at[j]], add=True)
        return ()
    lax.fori_loop(0, NCHUNK, deg_chunk, ())
    plsc.subcore_barrier()

    exchange(0)

    def coeffs(i, _):
        y = _rsqrt_rows(rows_a[i, :] + rows_b[i, :])
        h0r = c4_v[i, :]
        dis_v[i, :] = y
        c1_v[i, :] = (1.0 - ALPHA) * y * y
        c4_v[i, :] = ALPHA * h0r
        rows_a[i, :] = y * h0r
        return ()
    lax.fori_loop(0, STRIPE, coeffs, (), unroll=4)

    def seed_round(k_is_last):
        del k_is_last
        pltpu.sync_copy(rows_a.at[rS], g_sp.at[stripe])

        @pl.when(c == 0)
        def _():
            pltpu.sync_copy(rows_a.at[rS], s_sp.at[stripe])

        @pl.when(c != 0)
        def _():
            pltpu.sync_copy(zero_v, s_sp.at[stripe])
        plsc.subcore_barrier()

    seed_round(False)

    def start_gather(j, b):
        pltpu.async_copy(g_sp.at[src_v.at[j]], bufs[b], gsems[b])

    def wait_gather(b):
        pltpu.make_async_copy(g_sp.at[src_v.at[0]], bufs[b], gsems[b]).wait()

    def start_scatter(j, b):
        pltpu.async_copy(bufs[b], s_sp.at[dst_v.at[j]], ssems[b], add=True)

    def wait_scatter(b):
        pltpu.make_async_copy(bufs[b], s_sp.at[dst_v.at[0]], ssems[b]).wait()

    def edge_pipe(jj, _):
        j = jj * 2
        wait_gather(0)
        start_scatter(j, 0)
        start_gather(j + 1, 1)
        wait_gather(1)
        start_scatter(j + 1, 1)
        wait_scatter(0)

        @pl.when(jj < NCHUNK // 2 - 1)
        def _():
            start_gather(j + 2, 0)
        wait_scatter(1)
        return ()

    def update(i, _):
        rows_a[i, :] = (c1_v[i, :] * (rows_a[i, :] + rows_b[i, :])
                        + dis_v[i, :] * c4_v[i, :])
        return ()

    def final(i, _):
        rows_a[i, :] = ((1.0 - ALPHA) * dis_v[i, :]
                        * (rows_a[i, :] + rows_b[i, :]) + c4_v[i, :])
        return ()

    for k in range(K_PROP):
        start_gather(0, 0)
        lax.fori_loop(0, NCHUNK // 2, edge_pipe, ())
        plsc.subcore_barrier()

        exchange(k + 1)
        if k < K_PROP - 1:
            lax.fori_loop(0, STRIPE, update, (), unroll=8)
            seed_round(False)
        else:
            lax.fori_loop(0, STRIPE, final, (), unroll=8)

            @pl.when((c == 0) & (t < NS - 1))
            def _():
                pltpu.sync_copy(rows_a.at[rS], out_hbm.at[stripe])

            @pl.when((c == 0) & (t == NS - 1))
            def _():
                pltpu.sync_copy(rows_a.at[pl.ds(0, LAST_H)],
                                out_hbm.at[pl.ds((NS - 1) * STRIPE, LAST_H)])


@jax.jit
def _run(x, src, dst, W1, b1, W2, b2):
    h0 = _mlp(x, W1, b1, W2, b2)

    appnp = pl.kernel(
        _appnp_body,
        out_type=(
            jax.ShapeDtypeStruct((N_NODES, D), jnp.float32),
            jax.ShapeDtypeStruct((NC, 2, N_PAD, D), jnp.float32),
        ),
        mesh=_MESH,
        compiler_params=_SC_PARAMS,
        scratch_types=[
            pltpu.VMEM((NCHUNK, CHUNK), jnp.int32),
            pltpu.VMEM((NCHUNK, CHUNK), jnp.int32),
            pltpu.VMEM((CHUNK, D), jnp.float32),
            pltpu.VMEM((CHUNK, D), jnp.float32),
            pltpu.VMEM((STRIPE, D), jnp.float32),
            pltpu.VMEM((STRIPE, D), jnp.float32),
            pltpu.VMEM((STRIPE, D), jnp.float32),
            pltpu.VMEM((STRIPE, D), jnp.float32),
            pltpu.VMEM_SHARED((N_PAD, D), jnp.float32),
            pltpu.VMEM_SHARED((N_PAD, D), jnp.float32),
            pltpu.SemaphoreType.DMA,
            pltpu.SemaphoreType.DMA,
            pltpu.SemaphoreType.DMA,
            pltpu.SemaphoreType.DMA,
            pltpu.SemaphoreType.REGULAR,
        ],
    )
    out, _ = appnp(h0, src, dst)
    return out


def kernel(x, edge_index, epoch, W1, b1, W2, b2):
    src = edge_index[0].astype(jnp.int32).reshape(NW, NCHUNK, CHUNK)
    dst = edge_index[1].astype(jnp.int32).reshape(NW, NCHUNK, CHUNK)
    return _run(x, src, dst, W1, b1, W2, b2)
